# baseline (speedup 1.0000x reference)
import functools

import jax
import jax.numpy as jnp
from jax import lax
from jax.experimental import pallas as pl
from jax.experimental.pallas import tpu as pltpu
from jax.experimental.pallas import tpu_sc as plsc

N_FEATURES = 26
N_CATEGORIES = 1000
EMBED_DIM = 64
HIDDEN = 512
BATCH = 4096
IN_DIM = N_FEATURES * EMBED_DIM
TOTAL_ROWS = BATCH * N_FEATURES

_SPLIT = 2
_SLICE_B = BATCH // _SPLIT
_SLICE_ROWS = TOTAL_ROWS // _SPLIT

_NUM_CORES = 2
_NUM_SUBCORES = 16
_NW = _NUM_CORES * _NUM_SUBCORES
_CHUNK = 128


_NBUF = 4
_LOOKAHEAD = 3


def _sc_gather_body(chunks_per_w, idx_hbm, table_hbm, out_hbm,
                    idx_v, bufs, gsems, ssems):
    wid = lax.axis_index("s") * _NUM_CORES + lax.axis_index("c")
    base = wid * chunks_per_w
    rows_per_w = chunks_per_w * _CHUNK
    pltpu.sync_copy(idx_hbm.at[pl.ds(wid * rows_per_w, rows_per_w)], idx_v)

    def start_gather(c):
        pltpu.async_copy(table_hbm.at[idx_v.at[pl.ds(c * _CHUNK, _CHUNK)]],
                         bufs[c % _NBUF], gsems[c % _NBUF])

    def wait_gather(c):
        pltpu.make_async_copy(table_hbm.at[idx_v.at[pl.ds(c * _CHUNK, _CHUNK)]],
                              bufs[c % _NBUF], gsems[c % _NBUF]).wait()

    def start_store(c):
        off = pl.multiple_of((base + c) * _CHUNK, _CHUNK)
        pltpu.async_copy(bufs[c % _NBUF], out_hbm.at[pl.ds(off, _CHUNK)],
                         ssems[c % _NBUF])

    def wait_store(c):
        off = pl.multiple_of((base + c) * _CHUNK, _CHUNK)
        pltpu.make_async_copy(bufs[c % _NBUF], out_hbm.at[pl.ds(off, _CHUNK)],
                              ssems[c % _NBUF]).wait()

    for c in range(min(_LOOKAHEAD, chunks_per_w)):
        start_gather(c)
    for c in range(chunks_per_w):
        n = c + _LOOKAHEAD
        if n < chunks_per_w:
            if n - _NBUF >= 0:
                wait_store(n - _NBUF)
            start_gather(n)
        wait_gather(c)
        start_store(c)
    for c in range(max(0, chunks_per_w - _NBUF), chunks_per_w):
        wait_store(c)


@functools.lru_cache(maxsize=2)
def _sc_gather_fn(total_rows):
    chunks_per_w = total_rows // (_NW * _CHUNK)
    mesh = plsc.VectorSubcoreMesh(
        core_axis_name="c", subcore_axis_name="s",
        num_cores=_NUM_CORES, num_subcores=_NUM_SUBCORES,
    )
    return pl.kernel(
        functools.partial(_sc_gather_body, chunks_per_w),
        out_type=jax.ShapeDtypeStruct((total_rows, EMBED_DIM), jnp.bfloat16),
        mesh=mesh,
        scratch_types=[
            pltpu.VMEM((chunks_per_w * _CHUNK,), jnp.int32),
            [pltpu.VMEM((_CHUNK, EMBED_DIM), jnp.bfloat16)] * _NBUF,
            [pltpu.SemaphoreType.DMA] * _NBUF,
            [pltpu.SemaphoreType.DMA] * _NBUF,
        ],
        compiler_params=pltpu.CompilerParams(use_tc_tiling_on_sc=False),
    )


_BT = 512


def _mlp_body(h_ref, w0_ref, b0_ref, w1_ref, b1_ref, w2_ref, b2_ref,
              wout_ref, bout_ref, out_ref):
    h = h_ref[...]
    a = jnp.dot(h, w0_ref[...], preferred_element_type=jnp.float32)
    a = jnp.maximum(a + b0_ref[...], 0.0)
    a = jnp.dot(a, w1_ref[...], preferred_element_type=jnp.float32)
    a = jnp.maximum(a + b1_ref[...], 0.0)
    a = jnp.dot(a, w2_ref[...], preferred_element_type=jnp.float32)
    a = jnp.maximum(a + b2_ref[...], 0.0)
    z = jnp.sum(a * wout_ref[...], axis=1, keepdims=True) + bout_ref[...]
    out_ref[...] = 1.0 / (1.0 + jnp.exp(-z))


def _mlp(h, W0, b0, W1, b1, W2, b2, WoutT, bout):
    nb = h.shape[0]
    return pl.pallas_call(
        _mlp_body,
        grid=(nb // _BT,),
        in_specs=[
            pl.BlockSpec((_BT, IN_DIM), lambda i: (i, 0)),
            pl.BlockSpec((IN_DIM, HIDDEN), lambda i: (0, 0)),
            pl.BlockSpec((1, HIDDEN), lambda i: (0, 0)),
            pl.BlockSpec((HIDDEN, HIDDEN), lambda i: (0, 0)),
            pl.BlockSpec((1, HIDDEN), lambda i: (0, 0)),
            pl.BlockSpec((HIDDEN, HIDDEN), lambda i: (0, 0)),
            pl.BlockSpec((1, HIDDEN), lambda i: (0, 0)),
            pl.BlockSpec((1, HIDDEN), lambda i: (0, 0)),
            pl.BlockSpec((1, 1), lambda i: (0, 0)),
        ],
        out_specs=pl.BlockSpec((_BT, 1), lambda i: (i, 0)),
        out_shape=jax.ShapeDtypeStruct((nb, 1), jnp.float32),
    )(h, W0, b0, W1, b1, W2, b2, WoutT, bout)


def kernel(x, table, W0, b0, W1, b1, W2, b2, Wout, bout):
    offsets = (jnp.arange(N_FEATURES, dtype=jnp.int32) * N_CATEGORIES)
    idx = x.astype(jnp.int32) + offsets[None, :]
    idx4 = idx.reshape(_SPLIT, _SLICE_ROWS)

    table_bf = table.astype(jnp.bfloat16)
    W0b = W0.astype(jnp.bfloat16)
    gather = _sc_gather_fn(_SLICE_ROWS)
    b0r, b1r, b2r = (b.reshape(1, HIDDEN) for b in (b0, b1, b2))
    WoutT = Wout.reshape(1, HIDDEN)
    boutr = bout.reshape(1, 1)

    outs = []
    for s in range(_SPLIT):
        emb = gather(idx4[s], table_bf)
        h = emb.reshape(_SLICE_B, IN_DIM)
        outs.append(_mlp(h, W0b, b0r, W1, b1r, W2, b2r, WoutT, boutr))
    return jnp.concatenate(outs, axis=0)


# speedup vs baseline: 1.2566x; 1.2566x over previous
import functools

import jax
import jax.numpy as jnp
from jax import lax
from jax.experimental import pallas as pl
from jax.experimental.pallas import tpu as pltpu
from jax.experimental.pallas import tpu_sc as plsc

N_FEATURES = 26
N_CATEGORIES = 1000
EMBED_DIM = 64
HIDDEN = 512
BATCH = 4096
IN_DIM = N_FEATURES * EMBED_DIM
TOTAL_ROWS = BATCH * N_FEATURES

_SPLIT = 4
_SLICE_B = BATCH // _SPLIT
_SLICE_ROWS = TOTAL_ROWS // _SPLIT

_NUM_CORES = 2
_NUM_SUBCORES = 16
_NW = _NUM_CORES * _NUM_SUBCORES
_CHUNK = 128


_NBUF = 4
_LOOKAHEAD = 3


def _sc_gather_body(chunks_per_w, chunk, idx_hbm, table_hbm, out_hbm,
                    idx_v, bufs, gsems, ssems):
    wid = lax.axis_index("s") * _NUM_CORES + lax.axis_index("c")
    base = wid * chunks_per_w
    rows_per_w = chunks_per_w * chunk
    pltpu.sync_copy(idx_hbm.at[pl.ds(wid * rows_per_w, rows_per_w)], idx_v)

    def start_gather(c):
        pltpu.async_copy(table_hbm.at[idx_v.at[pl.ds(c * chunk, chunk)]],
                         bufs[c % _NBUF], gsems[c % _NBUF])

    def wait_gather(c):
        pltpu.make_async_copy(table_hbm.at[idx_v.at[pl.ds(c * chunk, chunk)]],
                              bufs[c % _NBUF], gsems[c % _NBUF]).wait()

    def start_store(c):
        off = pl.multiple_of((base + c) * chunk, chunk)
        pltpu.async_copy(bufs[c % _NBUF], out_hbm.at[pl.ds(off, chunk)],
                         ssems[c % _NBUF])

    def wait_store(c):
        off = pl.multiple_of((base + c) * chunk, chunk)
        pltpu.make_async_copy(bufs[c % _NBUF], out_hbm.at[pl.ds(off, chunk)],
                              ssems[c % _NBUF]).wait()

    for c in range(min(_LOOKAHEAD, chunks_per_w)):
        start_gather(c)
    for c in range(chunks_per_w):
        n = c + _LOOKAHEAD
        if n < chunks_per_w:
            if n - _NBUF >= 0:
                wait_store(n - _NBUF)
            start_gather(n)
        wait_gather(c)
        start_store(c)
    for c in range(max(0, chunks_per_w - _NBUF), chunks_per_w):
        wait_store(c)


@functools.lru_cache(maxsize=2)
def _sc_gather_fn(total_rows):
    rows_per_w = total_rows // _NW
    chunk = max(c for c in range(8, _CHUNK + 1, 8) if rows_per_w % c == 0)
    chunks_per_w = rows_per_w // chunk
    mesh = plsc.VectorSubcoreMesh(
        core_axis_name="c", subcore_axis_name="s",
        num_cores=_NUM_CORES, num_subcores=_NUM_SUBCORES,
    )
    return pl.kernel(
        functools.partial(_sc_gather_body, chunks_per_w, chunk),
        out_type=jax.ShapeDtypeStruct((total_rows, EMBED_DIM), jnp.float32),
        mesh=mesh,
        scratch_types=[
            pltpu.VMEM((rows_per_w,), jnp.int32),
            [pltpu.VMEM((chunk, EMBED_DIM), jnp.float32)] * _NBUF,
            [pltpu.SemaphoreType.DMA] * _NBUF,
            [pltpu.SemaphoreType.DMA] * _NBUF,
        ],
        compiler_params=pltpu.CompilerParams(use_tc_tiling_on_sc=False),
    )


_BT = 512


def _mlp_body(h_ref, w0_ref, b0_ref, w1_ref, b1_ref, w2_ref, b2_ref,
              wout_ref, bout_ref, out_ref):
    h = h_ref[...]
    a = jnp.dot(h, w0_ref[...], preferred_element_type=jnp.float32)
    a = jnp.maximum(a + b0_ref[...], 0.0)
    a = jnp.dot(a, w1_ref[...], preferred_element_type=jnp.float32)
    a = jnp.maximum(a + b1_ref[...], 0.0)
    a = jnp.dot(a, w2_ref[...], preferred_element_type=jnp.float32)
    a = jnp.maximum(a + b2_ref[...], 0.0)
    z = jnp.sum(a * wout_ref[...], axis=1, keepdims=True) + bout_ref[...]
    out_ref[...] = 1.0 / (1.0 + jnp.exp(-z))


def _mlp(h, W0, b0, W1, b1, W2, b2, WoutT, bout):
    nb = h.shape[0]
    return pl.pallas_call(
        _mlp_body,
        grid=(nb // _BT,),
        in_specs=[
            pl.BlockSpec((_BT, IN_DIM), lambda i: (i, 0)),
            pl.BlockSpec((IN_DIM, HIDDEN), lambda i: (0, 0)),
            pl.BlockSpec((1, HIDDEN), lambda i: (0, 0)),
            pl.BlockSpec((HIDDEN, HIDDEN), lambda i: (0, 0)),
            pl.BlockSpec((1, HIDDEN), lambda i: (0, 0)),
            pl.BlockSpec((HIDDEN, HIDDEN), lambda i: (0, 0)),
            pl.BlockSpec((1, HIDDEN), lambda i: (0, 0)),
            pl.BlockSpec((1, HIDDEN), lambda i: (0, 0)),
            pl.BlockSpec((1, 1), lambda i: (0, 0)),
        ],
        out_specs=pl.BlockSpec((_BT, 1), lambda i: (i, 0)),
        out_shape=jax.ShapeDtypeStruct((nb, 1), jnp.float32),
    )(h, W0, b0, W1, b1, W2, b2, WoutT, bout)


def kernel(x, table, W0, b0, W1, b1, W2, b2, Wout, bout):
    offsets = (jnp.arange(N_FEATURES, dtype=jnp.int32) * N_CATEGORIES)
    idx = x.astype(jnp.int32) + offsets[None, :]
    idx4 = idx.reshape(_SPLIT, _SLICE_ROWS)

    gather = _sc_gather_fn(_SLICE_ROWS)
    b0r, b1r, b2r = (b.reshape(1, HIDDEN) for b in (b0, b1, b2))
    WoutT = Wout.reshape(1, HIDDEN)
    boutr = bout.reshape(1, 1)

    outs = []
    for s in range(_SPLIT):
        emb = gather(idx4[s], table)
        h = emb.reshape(_SLICE_B, IN_DIM)
        outs.append(_mlp(h, W0, b0r, W1, b1r, W2, b2r, WoutT, boutr))
    return jnp.concatenate(outs, axis=0)


# speedup vs baseline: 1.3976x; 1.1122x over previous
import functools

import jax
import jax.numpy as jnp
from jax import lax
from jax.experimental import pallas as pl
from jax.experimental.pallas import tpu as pltpu
from jax.experimental.pallas import tpu_sc as plsc

N_FEATURES = 26
N_CATEGORIES = 1000
EMBED_DIM = 64
HIDDEN = 512
BATCH = 4096
IN_DIM = N_FEATURES * EMBED_DIM
TOTAL_ROWS = BATCH * N_FEATURES

_SPLIT = 1
_SLICE_B = BATCH // _SPLIT
_SLICE_ROWS = TOTAL_ROWS // _SPLIT

_NUM_CORES = 2
_NUM_SUBCORES = 16
_NW = _NUM_CORES * _NUM_SUBCORES
_CHUNK = 128


_NBUF = 4
_LOOKAHEAD = 3


def _sc_gather_body(chunks_per_w, chunk, idx_hbm, table_hbm, out_hbm,
                    idx_v, bufs, gsems, ssems):
    wid = lax.axis_index("s") * _NUM_CORES + lax.axis_index("c")
    base = wid * chunks_per_w
    rows_per_w = chunks_per_w * chunk
    pltpu.sync_copy(idx_hbm.at[pl.ds(wid * rows_per_w, rows_per_w)], idx_v)

    def start_gather(c):
        pltpu.async_copy(table_hbm.at[idx_v.at[pl.ds(c * chunk, chunk)]],
                         bufs[c % _NBUF], gsems[c % _NBUF])

    def wait_gather(c):
        pltpu.make_async_copy(table_hbm.at[idx_v.at[pl.ds(c * chunk, chunk)]],
                              bufs[c % _NBUF], gsems[c % _NBUF]).wait()

    def start_store(c):
        off = pl.multiple_of((base + c) * chunk, chunk)
        pltpu.async_copy(bufs[c % _NBUF], out_hbm.at[pl.ds(off, chunk)],
                         ssems[c % _NBUF])

    def wait_store(c):
        off = pl.multiple_of((base + c) * chunk, chunk)
        pltpu.make_async_copy(bufs[c % _NBUF], out_hbm.at[pl.ds(off, chunk)],
                              ssems[c % _NBUF]).wait()

    for c in range(min(_LOOKAHEAD, chunks_per_w)):
        start_gather(c)
    for c in range(chunks_per_w):
        n = c + _LOOKAHEAD
        if n < chunks_per_w:
            if n - _NBUF >= 0:
                wait_store(n - _NBUF)
            start_gather(n)
        wait_gather(c)
        start_store(c)
    for c in range(max(0, chunks_per_w - _NBUF), chunks_per_w):
        wait_store(c)


@functools.lru_cache(maxsize=2)
def _sc_gather_fn(total_rows):
    rows_per_w = total_rows // _NW
    chunk = max(c for c in range(8, _CHUNK + 1, 8) if rows_per_w % c == 0)
    chunks_per_w = rows_per_w // chunk
    mesh = plsc.VectorSubcoreMesh(
        core_axis_name="c", subcore_axis_name="s",
        num_cores=_NUM_CORES, num_subcores=_NUM_SUBCORES,
    )
    return pl.kernel(
        functools.partial(_sc_gather_body, chunks_per_w, chunk),
        out_type=jax.ShapeDtypeStruct((total_rows, EMBED_DIM), jnp.float32),
        mesh=mesh,
        scratch_types=[
            pltpu.VMEM((rows_per_w,), jnp.int32),
            [pltpu.VMEM((chunk, EMBED_DIM), jnp.float32)] * _NBUF,
            [pltpu.SemaphoreType.DMA] * _NBUF,
            [pltpu.SemaphoreType.DMA] * _NBUF,
        ],
        compiler_params=pltpu.CompilerParams(use_tc_tiling_on_sc=False),
    )


_BT = 512


def _mlp_body(h_ref, w0_ref, b0_ref, w1_ref, b1_ref, w2_ref, b2_ref,
              wout_ref, bout_ref, out_ref):
    h = h_ref[...]
    a = jnp.dot(h, w0_ref[...], preferred_element_type=jnp.float32)
    a = jnp.maximum(a + b0_ref[...], 0.0)
    a = jnp.dot(a, w1_ref[...], preferred_element_type=jnp.float32)
    a = jnp.maximum(a + b1_ref[...], 0.0)
    a = jnp.dot(a, w2_ref[...], preferred_element_type=jnp.float32)
    a = jnp.maximum(a + b2_ref[...], 0.0)
    z = jnp.sum(a * wout_ref[...], axis=1, keepdims=True) + bout_ref[...]
    out_ref[...] = 1.0 / (1.0 + jnp.exp(-z))


def _mlp(h, W0, b0, W1, b1, W2, b2, WoutT, bout):
    nb = h.shape[0]
    return pl.pallas_call(
        _mlp_body,
        grid=(nb // _BT,),
        in_specs=[
            pl.BlockSpec((_BT, IN_DIM), lambda i: (i, 0)),
            pl.BlockSpec((IN_DIM, HIDDEN), lambda i: (0, 0)),
            pl.BlockSpec((1, HIDDEN), lambda i: (0, 0)),
            pl.BlockSpec((HIDDEN, HIDDEN), lambda i: (0, 0)),
            pl.BlockSpec((1, HIDDEN), lambda i: (0, 0)),
            pl.BlockSpec((HIDDEN, HIDDEN), lambda i: (0, 0)),
            pl.BlockSpec((1, HIDDEN), lambda i: (0, 0)),
            pl.BlockSpec((1, HIDDEN), lambda i: (0, 0)),
            pl.BlockSpec((1, 1), lambda i: (0, 0)),
        ],
        out_specs=pl.BlockSpec((_BT, 1), lambda i: (i, 0)),
        out_shape=jax.ShapeDtypeStruct((nb, 1), jnp.float32),
    )(h, W0, b0, W1, b1, W2, b2, WoutT, bout)


def kernel(x, table, W0, b0, W1, b1, W2, b2, Wout, bout):
    offsets = (jnp.arange(N_FEATURES, dtype=jnp.int32) * N_CATEGORIES)
    idx = x.astype(jnp.int32) + offsets[None, :]
    idx4 = idx.reshape(_SPLIT, _SLICE_ROWS)

    gather = _sc_gather_fn(_SLICE_ROWS)
    b0r, b1r, b2r = (b.reshape(1, HIDDEN) for b in (b0, b1, b2))
    WoutT = Wout.reshape(1, HIDDEN)
    boutr = bout.reshape(1, 1)

    outs = []
    for s in range(_SPLIT):
        emb = gather(idx4[s], table)
        h = emb.reshape(_SLICE_B, IN_DIM)
        outs.append(_mlp(h, W0, b0r, W1, b1r, W2, b2r, WoutT, boutr))
    return jnp.concatenate(outs, axis=0)


# speedup vs baseline: 1.4013x; 1.0027x over previous
import functools

import jax
import jax.numpy as jnp
from jax import lax
from jax.experimental import pallas as pl
from jax.experimental.pallas import tpu as pltpu
from jax.experimental.pallas import tpu_sc as plsc

N_FEATURES = 26
N_CATEGORIES = 1000
EMBED_DIM = 64
HIDDEN = 512
BATCH = 4096
IN_DIM = N_FEATURES * EMBED_DIM
TOTAL_ROWS = BATCH * N_FEATURES

_SPLIT = 1
_SLICE_B = BATCH // _SPLIT
_SLICE_ROWS = TOTAL_ROWS // _SPLIT

_NUM_CORES = 2
_NUM_SUBCORES = 16
_NW = _NUM_CORES * _NUM_SUBCORES
_CHUNK = 128


_NBUF = 4
_LOOKAHEAD = 3


def _sc_gather_body(chunks_per_w, chunk, idx_hbm, table_hbm, out_hbm,
                    idx_v, bufs, gsems, ssems):
    wid = lax.axis_index("s") * _NUM_CORES + lax.axis_index("c")
    base = wid * chunks_per_w
    rows_per_w = chunks_per_w * chunk
    pltpu.sync_copy(idx_hbm.at[pl.ds(wid * rows_per_w, rows_per_w)], idx_v)

    def start_gather(c):
        pltpu.async_copy(table_hbm.at[idx_v.at[pl.ds(c * chunk, chunk)]],
                         bufs[c % _NBUF], gsems[c % _NBUF])

    def wait_gather(c):
        pltpu.make_async_copy(table_hbm.at[idx_v.at[pl.ds(c * chunk, chunk)]],
                              bufs[c % _NBUF], gsems[c % _NBUF]).wait()

    def start_store(c):
        off = pl.multiple_of((base + c) * chunk, chunk)
        pltpu.async_copy(bufs[c % _NBUF], out_hbm.at[pl.ds(off, chunk)],
                         ssems[c % _NBUF])

    def wait_store(c):
        off = pl.multiple_of((base + c) * chunk, chunk)
        pltpu.make_async_copy(bufs[c % _NBUF], out_hbm.at[pl.ds(off, chunk)],
                              ssems[c % _NBUF]).wait()

    for c in range(min(_LOOKAHEAD, chunks_per_w)):
        start_gather(c)
    for c in range(chunks_per_w):
        n = c + _LOOKAHEAD
        if n < chunks_per_w:
            if n - _NBUF >= 0:
                wait_store(n - _NBUF)
            start_gather(n)
        wait_gather(c)
        start_store(c)
    for c in range(max(0, chunks_per_w - _NBUF), chunks_per_w):
        wait_store(c)


@functools.lru_cache(maxsize=2)
def _sc_gather_fn(total_rows):
    rows_per_w = total_rows // _NW
    chunk = max(c for c in range(8, _CHUNK + 1, 8) if rows_per_w % c == 0)
    chunks_per_w = rows_per_w // chunk
    mesh = plsc.VectorSubcoreMesh(
        core_axis_name="c", subcore_axis_name="s",
        num_cores=_NUM_CORES, num_subcores=_NUM_SUBCORES,
    )
    return pl.kernel(
        functools.partial(_sc_gather_body, chunks_per_w, chunk),
        out_type=jax.ShapeDtypeStruct((total_rows, EMBED_DIM), jnp.float32),
        mesh=mesh,
        scratch_types=[
            pltpu.VMEM((rows_per_w,), jnp.int32),
            [pltpu.VMEM((chunk, EMBED_DIM), jnp.float32)] * _NBUF,
            [pltpu.SemaphoreType.DMA] * _NBUF,
            [pltpu.SemaphoreType.DMA] * _NBUF,
        ],
        compiler_params=pltpu.CompilerParams(use_tc_tiling_on_sc=False),
    )


_BT = 512


def _mlp_body(h_ref, w0_ref, b0_ref, w1_ref, b1_ref, w2_ref, b2_ref,
              wout_ref, bout_ref, out_ref):
    bf = jnp.bfloat16
    h = h_ref[...].astype(bf)
    a = jnp.dot(h, w0_ref[...].astype(bf), preferred_element_type=jnp.float32)
    a = jnp.maximum(a + b0_ref[...], 0.0)
    a = jnp.dot(a.astype(bf), w1_ref[...].astype(bf),
                preferred_element_type=jnp.float32)
    a = jnp.maximum(a + b1_ref[...], 0.0)
    a = jnp.dot(a.astype(bf), w2_ref[...].astype(bf),
                preferred_element_type=jnp.float32)
    a = jnp.maximum(a + b2_ref[...], 0.0)
    z = jnp.sum(a * wout_ref[...], axis=1, keepdims=True) + bout_ref[...]
    out_ref[...] = 1.0 / (1.0 + jnp.exp(-z))


def _mlp(h, W0, b0, W1, b1, W2, b2, WoutT, bout):
    nb = h.shape[0]
    return pl.pallas_call(
        _mlp_body,
        grid=(nb // _BT,),
        in_specs=[
            pl.BlockSpec((_BT, IN_DIM), lambda i: (i, 0)),
            pl.BlockSpec((IN_DIM, HIDDEN), lambda i: (0, 0)),
            pl.BlockSpec((1, HIDDEN), lambda i: (0, 0)),
            pl.BlockSpec((HIDDEN, HIDDEN), lambda i: (0, 0)),
            pl.BlockSpec((1, HIDDEN), lambda i: (0, 0)),
            pl.BlockSpec((HIDDEN, HIDDEN), lambda i: (0, 0)),
            pl.BlockSpec((1, HIDDEN), lambda i: (0, 0)),
            pl.BlockSpec((1, HIDDEN), lambda i: (0, 0)),
            pl.BlockSpec((1, 1), lambda i: (0, 0)),
        ],
        out_specs=pl.BlockSpec((_BT, 1), lambda i: (i, 0)),
        out_shape=jax.ShapeDtypeStruct((nb, 1), jnp.float32),
    )(h, W0, b0, W1, b1, W2, b2, WoutT, bout)


def kernel(x, table, W0, b0, W1, b1, W2, b2, Wout, bout):
    offsets = (jnp.arange(N_FEATURES, dtype=jnp.int32) * N_CATEGORIES)
    idx = x.astype(jnp.int32) + offsets[None, :]
    idx4 = idx.reshape(_SPLIT, _SLICE_ROWS)

    gather = _sc_gather_fn(_SLICE_ROWS)
    b0r, b1r, b2r = (b.reshape(1, HIDDEN) for b in (b0, b1, b2))
    WoutT = Wout.reshape(1, HIDDEN)
    boutr = bout.reshape(1, 1)

    outs = []
    for s in range(_SPLIT):
        emb = gather(idx4[s], table)
        h = emb.reshape(_SLICE_B, IN_DIM)
        outs.append(_mlp(h, W0, b0r, W1, b1r, W2, b2r, WoutT, boutr))
    return jnp.concatenate(outs, axis=0)


# speedup vs baseline: 1.4111x; 1.0069x over previous
import functools

import jax
import jax.numpy as jnp
from jax import lax
from jax.experimental import pallas as pl
from jax.experimental.pallas import tpu as pltpu
from jax.experimental.pallas import tpu_sc as plsc

N_FEATURES = 26
N_CATEGORIES = 1000
EMBED_DIM = 64
HIDDEN = 512
BATCH = 4096
IN_DIM = N_FEATURES * EMBED_DIM
TOTAL_ROWS = BATCH * N_FEATURES

_SPLIT = 1
_SLICE_B = BATCH // _SPLIT
_SLICE_ROWS = TOTAL_ROWS // _SPLIT

_NUM_CORES = 2
_NUM_SUBCORES = 16
_NW = _NUM_CORES * _NUM_SUBCORES
_CHUNK = 128


_NBUF = 4
_LOOKAHEAD = 3


def _sc_gather_body(chunks_per_w, chunk, idx_hbm, table_hbm, out_hbm,
                    idx_v, bufs, gsems, ssems):
    wid = lax.axis_index("s") * _NUM_CORES + lax.axis_index("c")
    base = wid * chunks_per_w
    rows_per_w = chunks_per_w * chunk
    pltpu.sync_copy(idx_hbm.at[pl.ds(wid * rows_per_w, rows_per_w)], idx_v)

    def start_gather(c):
        pltpu.async_copy(table_hbm.at[idx_v.at[pl.ds(c * chunk, chunk)]],
                         bufs[c % _NBUF], gsems[c % _NBUF])

    def wait_gather(c):
        pltpu.make_async_copy(table_hbm.at[idx_v.at[pl.ds(c * chunk, chunk)]],
                              bufs[c % _NBUF], gsems[c % _NBUF]).wait()

    def start_store(c):
        off = pl.multiple_of((base + c) * chunk, chunk)
        pltpu.async_copy(bufs[c % _NBUF], out_hbm.at[pl.ds(off, chunk)],
                         ssems[c % _NBUF])

    def wait_store(c):
        off = pl.multiple_of((base + c) * chunk, chunk)
        pltpu.make_async_copy(bufs[c % _NBUF], out_hbm.at[pl.ds(off, chunk)],
                              ssems[c % _NBUF]).wait()

    for c in range(min(_LOOKAHEAD, chunks_per_w)):
        start_gather(c)
    for c in range(chunks_per_w):
        n = c + _LOOKAHEAD
        if n < chunks_per_w:
            if n - _NBUF >= 0:
                wait_store(n - _NBUF)
            start_gather(n)
        wait_gather(c)
        start_store(c)
    for c in range(max(0, chunks_per_w - _NBUF), chunks_per_w):
        wait_store(c)


@functools.lru_cache(maxsize=2)
def _sc_gather_fn(total_rows):
    rows_per_w = total_rows // _NW
    chunk = max(c for c in range(8, _CHUNK + 1, 8) if rows_per_w % c == 0)
    chunks_per_w = rows_per_w // chunk
    mesh = plsc.VectorSubcoreMesh(
        core_axis_name="c", subcore_axis_name="s",
        num_cores=_NUM_CORES, num_subcores=_NUM_SUBCORES,
    )
    return pl.kernel(
        functools.partial(_sc_gather_body, chunks_per_w, chunk),
        out_type=jax.ShapeDtypeStruct((total_rows, EMBED_DIM), jnp.float32),
        mesh=mesh,
        scratch_types=[
            pltpu.VMEM((rows_per_w,), jnp.int32),
            [pltpu.VMEM((chunk, EMBED_DIM), jnp.float32)] * _NBUF,
            [pltpu.SemaphoreType.DMA] * _NBUF,
            [pltpu.SemaphoreType.DMA] * _NBUF,
        ],
        compiler_params=pltpu.CompilerParams(use_tc_tiling_on_sc=False),
    )


_BT = 1024


def _mlp_body(h_ref, w0_ref, b0_ref, w1_ref, b1_ref, w2_ref, b2_ref,
              wout_ref, bout_ref, out_ref):
    h = h_ref[...]
    a = jnp.dot(h, w0_ref[...], preferred_element_type=jnp.float32)
    a = jnp.maximum(a + b0_ref[...], 0.0)
    a = jnp.dot(a, w1_ref[...], preferred_element_type=jnp.float32)
    a = jnp.maximum(a + b1_ref[...], 0.0)
    a = jnp.dot(a, w2_ref[...], preferred_element_type=jnp.float32)
    a = jnp.maximum(a + b2_ref[...], 0.0)
    z = jnp.sum(a * wout_ref[...], axis=1, keepdims=True) + bout_ref[...]
    out_ref[...] = 1.0 / (1.0 + jnp.exp(-z))


def _mlp(h, W0, b0, W1, b1, W2, b2, WoutT, bout):
    nb = h.shape[0]
    return pl.pallas_call(
        _mlp_body,
        grid=(nb // _BT,),
        in_specs=[
            pl.BlockSpec((_BT, IN_DIM), lambda i: (i, 0)),
            pl.BlockSpec((IN_DIM, HIDDEN), lambda i: (0, 0)),
            pl.BlockSpec((1, HIDDEN), lambda i: (0, 0)),
            pl.BlockSpec((HIDDEN, HIDDEN), lambda i: (0, 0)),
            pl.BlockSpec((1, HIDDEN), lambda i: (0, 0)),
            pl.BlockSpec((HIDDEN, HIDDEN), lambda i: (0, 0)),
            pl.BlockSpec((1, HIDDEN), lambda i: (0, 0)),
            pl.BlockSpec((1, HIDDEN), lambda i: (0, 0)),
            pl.BlockSpec((1, 1), lambda i: (0, 0)),
        ],
        out_specs=pl.BlockSpec((_BT, 1), lambda i: (i, 0)),
        out_shape=jax.ShapeDtypeStruct((nb, 1), jnp.float32),
    )(h, W0, b0, W1, b1, W2, b2, WoutT, bout)


def kernel(x, table, W0, b0, W1, b1, W2, b2, Wout, bout):
    offsets = (jnp.arange(N_FEATURES, dtype=jnp.int32) * N_CATEGORIES)
    idx = x.astype(jnp.int32) + offsets[None, :]
    idx4 = idx.reshape(_SPLIT, _SLICE_ROWS)

    gather = _sc_gather_fn(_SLICE_ROWS)
    b0r, b1r, b2r = (b.reshape(1, HIDDEN) for b in (b0, b1, b2))
    WoutT = Wout.reshape(1, HIDDEN)
    boutr = bout.reshape(1, 1)

    outs = []
    for s in range(_SPLIT):
        emb = gather(idx4[s], table)
        h = emb.reshape(_SLICE_B, IN_DIM)
        outs.append(_mlp(h, W0, b0r, W1, b1r, W2, b2r, WoutT, boutr))
    return jnp.concatenate(outs, axis=0)


# speedup vs baseline: 1.4243x; 1.0094x over previous
import functools

import jax
import jax.numpy as jnp
from jax import lax
from jax.experimental import pallas as pl
from jax.experimental.pallas import tpu as pltpu
from jax.experimental.pallas import tpu_sc as plsc

N_FEATURES = 26
N_CATEGORIES = 1000
EMBED_DIM = 64
HIDDEN = 512
BATCH = 4096
IN_DIM = N_FEATURES * EMBED_DIM
TOTAL_ROWS = BATCH * N_FEATURES

_SPLIT = 1
_SLICE_B = BATCH // _SPLIT
_SLICE_ROWS = TOTAL_ROWS // _SPLIT

_NUM_CORES = 2
_NUM_SUBCORES = 16
_NW = _NUM_CORES * _NUM_SUBCORES
_CHUNK = 128


_NBUF = 6
_LOOKAHEAD = 5


def _sc_gather_body(chunks_per_w, chunk, idx_hbm, table_hbm, out_hbm,
                    idx_v, bufs, gsems, ssems):
    wid = lax.axis_index("s") * _NUM_CORES + lax.axis_index("c")
    base = wid * chunks_per_w
    rows_per_w = chunks_per_w * chunk
    pltpu.sync_copy(idx_hbm.at[pl.ds(wid * rows_per_w, rows_per_w)], idx_v)

    def start_gather(c):
        pltpu.async_copy(table_hbm.at[idx_v.at[pl.ds(c * chunk, chunk)]],
                         bufs[c % _NBUF], gsems[c % _NBUF])

    def wait_gather(c):
        pltpu.make_async_copy(table_hbm.at[idx_v.at[pl.ds(c * chunk, chunk)]],
                              bufs[c % _NBUF], gsems[c % _NBUF]).wait()

    def start_store(c):
        off = pl.multiple_of((base + c) * chunk, chunk)
        pltpu.async_copy(bufs[c % _NBUF], out_hbm.at[pl.ds(off, chunk)],
                         ssems[c % _NBUF])

    def wait_store(c):
        off = pl.multiple_of((base + c) * chunk, chunk)
        pltpu.make_async_copy(bufs[c % _NBUF], out_hbm.at[pl.ds(off, chunk)],
                              ssems[c % _NBUF]).wait()

    for c in range(min(_LOOKAHEAD, chunks_per_w)):
        start_gather(c)
    for c in range(chunks_per_w):
        n = c + _LOOKAHEAD
        if n < chunks_per_w:
            if n - _NBUF >= 0:
                wait_store(n - _NBUF)
            start_gather(n)
        wait_gather(c)
        start_store(c)
    for c in range(max(0, chunks_per_w - _NBUF), chunks_per_w):
        wait_store(c)


@functools.lru_cache(maxsize=2)
def _sc_gather_fn(total_rows):
    rows_per_w = total_rows // _NW
    chunk = max(c for c in range(8, _CHUNK + 1, 8) if rows_per_w % c == 0)
    chunks_per_w = rows_per_w // chunk
    mesh = plsc.VectorSubcoreMesh(
        core_axis_name="c", subcore_axis_name="s",
        num_cores=_NUM_CORES, num_subcores=_NUM_SUBCORES,
    )
    return pl.kernel(
        functools.partial(_sc_gather_body, chunks_per_w, chunk),
        out_type=jax.ShapeDtypeStruct((total_rows, EMBED_DIM), jnp.float32),
        mesh=mesh,
        scratch_types=[
            pltpu.VMEM((rows_per_w,), jnp.int32),
            [pltpu.VMEM((chunk, EMBED_DIM), jnp.float32)] * _NBUF,
            [pltpu.SemaphoreType.DMA] * _NBUF,
            [pltpu.SemaphoreType.DMA] * _NBUF,
        ],
        compiler_params=pltpu.CompilerParams(use_tc_tiling_on_sc=False),
    )


_BT = 1024


def _mlp_body(h_ref, w0_ref, b0_ref, w1_ref, b1_ref, w2_ref, b2_ref,
              wout_ref, bout_ref, out_ref):
    h = h_ref[...]
    a = jnp.dot(h, w0_ref[...], preferred_element_type=jnp.float32)
    a = jnp.maximum(a + b0_ref[...], 0.0)
    a = jnp.dot(a, w1_ref[...], preferred_element_type=jnp.float32)
    a = jnp.maximum(a + b1_ref[...], 0.0)
    a = jnp.dot(a, w2_ref[...], preferred_element_type=jnp.float32)
    a = jnp.maximum(a + b2_ref[...], 0.0)
    z = jnp.sum(a * wout_ref[...], axis=1, keepdims=True) + bout_ref[...]
    out_ref[...] = 1.0 / (1.0 + jnp.exp(-z))


def _mlp(h, W0, b0, W1, b1, W2, b2, WoutT, bout):
    nb = h.shape[0]
    return pl.pallas_call(
        _mlp_body,
        grid=(nb // _BT,),
        in_specs=[
            pl.BlockSpec((_BT, IN_DIM), lambda i: (i, 0)),
            pl.BlockSpec((IN_DIM, HIDDEN), lambda i: (0, 0)),
            pl.BlockSpec((1, HIDDEN), lambda i: (0, 0)),
            pl.BlockSpec((HIDDEN, HIDDEN), lambda i: (0, 0)),
            pl.BlockSpec((1, HIDDEN), lambda i: (0, 0)),
            pl.BlockSpec((HIDDEN, HIDDEN), lambda i: (0, 0)),
            pl.BlockSpec((1, HIDDEN), lambda i: (0, 0)),
            pl.BlockSpec((1, HIDDEN), lambda i: (0, 0)),
            pl.BlockSpec((1, 1), lambda i: (0, 0)),
        ],
        out_specs=pl.BlockSpec((_BT, 1), lambda i: (i, 0)),
        out_shape=jax.ShapeDtypeStruct((nb, 1), jnp.float32),
    )(h, W0, b0, W1, b1, W2, b2, WoutT, bout)


def kernel(x, table, W0, b0, W1, b1, W2, b2, Wout, bout):
    offsets = (jnp.arange(N_FEATURES, dtype=jnp.int32) * N_CATEGORIES)
    idx = x.astype(jnp.int32) + offsets[None, :]
    idx4 = idx.reshape(_SPLIT, _SLICE_ROWS)

    gather = _sc_gather_fn(_SLICE_ROWS)
    b0r, b1r, b2r = (b.reshape(1, HIDDEN) for b in (b0, b1, b2))
    WoutT = Wout.reshape(1, HIDDEN)
    boutr = bout.reshape(1, 1)

    outs = []
    for s in range(_SPLIT):
        emb = gather(idx4[s], table)
        h = emb.reshape(_SLICE_B, IN_DIM)
        outs.append(_mlp(h, W0, b0r, W1, b1r, W2, b2r, WoutT, boutr))
    return jnp.concatenate(outs, axis=0)
